# baseline (speedup 1.0000x reference)
import functools

import jax
import jax.numpy as jnp
from jax import lax
from jax.experimental import pallas as pl
from jax.experimental.pallas import tpu as pltpu
from jax.experimental.pallas import tpu_sc as plsc

F32 = jnp.float32
N_NODES = 1778
OUT = 901
SELPAD = 1792
EPS = 1e-5
CNT = N_NODES * OUT

GRAPH_DIMS = {
    'drug': (2060, 1183, 688, 384),
    'inc': (2459, 1582, 824, 512),
    'mrna': (3929, 3052, 992, 512),
}
BM2 = 512
BM_CAM = 256
SC_B = 32768
SC_BPW = 1024


def _mm_body(a_ref, b_ref, d_ref, o_ref):
    a = a_ref[...].astype(BF16)
    b = b_ref[...].astype(BF16)
    acc = jnp.dot(a, b, preferred_element_type=F32)
    o_ref[...] = (acc * d_ref[:, 0:1]).astype(o_ref.dtype)


def _mm(x, w, dinv_b, bm, out_dtype=F32):
    n = x.shape[0]
    return pl.pallas_call(
        _mm_body,
        grid=(pl.cdiv(n, bm),),
        in_specs=[pl.BlockSpec((bm, n), lambda i: (i, 0)),
                  pl.BlockSpec((n, OUT), lambda i: (0, 0)),
                  pl.BlockSpec((bm, 128), lambda i: (i, 0))],
        out_specs=pl.BlockSpec((bm, OUT), lambda i: (i, 0)),
        out_shape=jax.ShapeDtypeStruct((n, OUT), out_dtype),
        compiler_params=pltpu.CompilerParams(
            dimension_semantics=("arbitrary",)),
    )(x, w, dinv_b)


def _colsum_body(a_ref, o_ref, *, nk, bk, nrows):
    k = pl.program_id(0)
    a = a_ref[...]
    rem = nrows - k * bk
    row = lax.broadcasted_iota(jnp.int32, a.shape, 0)
    a = jnp.where(row < rem, a, 0.0)
    part = jnp.broadcast_to(jnp.sum(a, axis=0, keepdims=True), o_ref.shape)

    @pl.when(k == 0)
    def _():
        o_ref[...] = part + 1.0

    @pl.when(k > 0)
    def _():
        o_ref[...] += part

    @pl.when(k == nk - 1)
    def _():
        o_ref[...] = lax.rsqrt(o_ref[...])


def _dinv(adj, bk=256):
    n = adj.shape[0]
    nk = pl.cdiv(n, bk)
    out = pl.pallas_call(
        functools.partial(_colsum_body, nk=nk, bk=bk, nrows=n),
        grid=(nk,),
        in_specs=[pl.BlockSpec((bk, n), lambda k: (k, 0))],
        out_specs=pl.BlockSpec((8, n), lambda k: (0, 0)),
        out_shape=jax.ShapeDtypeStruct((8, n), F32),
        compiler_params=pltpu.CompilerParams(
            dimension_semantics=("arbitrary",)),
    )(adj)
    return out[0]


def _agg_body(adj_ref, hs_ref, hself_ref, dc_ref, bias_ref, o_ref):
    a = adj_ref[...].astype(BF16)
    acc = lax.dot_general(
        a, hs_ref[...], (((0,), (0,)), ((), ())), preferred_element_type=F32)
    dc = dc_ref[:, 0:1]
    val = dc * (acc + hself_ref[...].astype(F32)) + bias_ref[0:1, :]
    o_ref[...] = jnp.maximum(val, 0.0).astype(BF16)


def _agg(adj, dinv_b, hs, bias8, bm2):
    n = adj.shape[0]
    return pl.pallas_call(
        _agg_body,
        grid=(pl.cdiv(n, bm2),),
        in_specs=[
            pl.BlockSpec((n, bm2), lambda i: (0, i)),
            pl.BlockSpec((n, OUT), lambda i: (0, 0)),
            pl.BlockSpec((bm2, OUT), lambda i: (i, 0)),
            pl.BlockSpec((bm2, 128), lambda i: (i, 0)),
            pl.BlockSpec((8, OUT), lambda i: (0, 0)),
        ],
        out_specs=pl.BlockSpec((bm2, OUT), lambda i: (i, 0)),
        out_shape=jax.ShapeDtypeStruct((n, OUT), BF16),
        compiler_params=pltpu.CompilerParams(
            dimension_semantics=("arbitrary",)),
    )(adj, hs, hs, dinv_b, bias8)


def _wc_body(w1_ref, w2_ref, w3_ref, w4_ref, o1_ref, o2_ref, o3_ref):
    t3 = jnp.dot(w3_ref[...], w4_ref[...], preferred_element_type=F32)
    t2 = jnp.dot(w2_ref[...], t3, preferred_element_type=F32)
    o1_ref[...] = jnp.dot(w1_ref[...], t2, preferred_element_type=F32)
    o2_ref[...] = t2
    o3_ref[...] = t3


def _wc_chain(w1, w2, w3, w4):
    full = lambda s: pl.BlockSpec(s, lambda: (0, 0))
    return pl.pallas_call(
        _wc_body,
        in_specs=[full((1802, 1024)), full((1024, 512)),
                  full((512, 64)), full((64, 1))],
        out_specs=(full((1802, 1)), full((1024, 1)), full((512, 1))),
        out_shape=(jax.ShapeDtypeStruct((1802, 1), F32),
                   jax.ShapeDtypeStruct((1024, 1), F32),
                   jax.ShapeDtypeStruct((512, 1), F32)),
    )(w1, w2, w3, w4)


def _lane_sums(vals):
    lane = lax.broadcasted_iota(jnp.int32, (8, 128), 1)
    out = jnp.zeros((8, 128), F32)
    for idx, s in enumerate(vals):
        out = out + jnp.where(lane == idx, s, 0.0)
    return out


def _chan(d_ref, i_ref, m_ref, f_ref):
    return (d_ref[...].astype(F32), i_ref[...].astype(F32),
            m_ref[...].astype(F32), f_ref[...].astype(F32))


def _p1_body(w_ref, d_ref, i_ref, m_ref, f_ref, o_ref):
    g = pl.program_id(0)
    d, i, m, f = _chan(d_ref, i_ref, m_ref, f_ref)
    tl = w_ref[0] * d + w_ref[1] * i + w_ref[2] * m + w_ref[3] * f
    tg = w_ref[4] * d + w_ref[5] * i + w_ref[6] * m + w_ref[7] * f
    row = g * BM_CAM + lax.broadcasted_iota(jnp.int32, tl.shape, 0)
    ok = row < N_NODES
    tl = jnp.where(ok, tl, 0.0)
    tg = jnp.where(ok, tg, 0.0)
    part = _lane_sums([jnp.sum(tl), jnp.sum(tl * tl),
                       jnp.sum(tg), jnp.sum(tg * tg)])

    @pl.when(g == 0)
    def _():
        o_ref[...] = part

    @pl.when(g > 0)
    def _():
        o_ref[...] += part


def _p2_body(s_ref, d_ref, i_ref, m_ref, f_ref, o_ref):
    g = pl.program_id(0)
    d, i, m, f = _chan(d_ref, i_ref, m_ref, f_ref)
    tl = s_ref[0] * d + s_ref[1] * i + s_ref[2] * m + s_ref[3] * f
    tg = s_ref[4] * d + s_ref[5] * i + s_ref[6] * m + s_ref[7] * f
    yl = jnp.maximum(s_ref[8] * tl + s_ref[9], 0.0)
    yg = jnp.maximum(s_ref[10] * tg + s_ref[11], 0.0)
    row = g * BM_CAM + lax.broadcasted_iota(jnp.int32, yl.shape, 0)
    ok = row < N_NODES
    yl = jnp.where(ok, yl, 0.0)
    yg = jnp.where(ok, yg, 0.0)
    part = _lane_sums([jnp.sum(yl), jnp.sum(yl * yl),
                       jnp.sum(yg), jnp.sum(yg * yg)])

    @pl.when(g == 0)
    def _():
        o_ref[...] = part

    @pl.when(g > 0)
    def _():
        o_ref[...] += part


def _p3_body(s_ref, wv_ref, d_ref, i_ref, m_ref, f_ref, o_ref):
    chans = _chan(d_ref, i_ref, m_ref, f_ref)
    d, i, m, f = chans
    tl = s_ref[0] * d + s_ref[1] * i + s_ref[2] * m + s_ref[3] * f
    tg = s_ref[4] * d + s_ref[5] * i + s_ref[6] * m + s_ref[7] * f
    yl = jnp.maximum(s_ref[8] * tl + s_ref[9], 0.0)
    yg = jnp.maximum(s_ref[10] * tg + s_ref[11], 0.0)
    sx = jnp.zeros_like(tl)
    for c in range(4):
        att = jax.nn.sigmoid(s_ref[12 + c] * yl + s_ref[16 + c] * yg
                             + s_ref[20 + c])
        sx = sx + chans[c] * att
    sx = sx * 0.25
    u = jnp.sum(sx * wv_ref[0:1, :], axis=1)
    v = jnp.sum(sx * wv_ref[1:2, :], axis=1)
    lane = lax.broadcasted_iota(jnp.int32, o_ref.shape, 1)
    o_ref[...] = (jnp.where(lane == 0, u[:, None], 0.0)
                  + jnp.where(lane == 1, v[:, None], 0.0))


def _cam_call(body, scal, maps, extra_vmem=None, out_rows=8,
              out_block_rows=None):
    n_scal = scal.shape[0]
    in_specs = [pl.BlockSpec(memory_space=pltpu.SMEM)]
    args = [scal]
    if extra_vmem is not None:
        in_specs.append(pl.BlockSpec((8, OUT), lambda g: (0, 0)))
        args.append(extra_vmem)
    for mp in maps:
        in_specs.append(pl.BlockSpec((BM_CAM, OUT), lambda g: (g, 0)))
        args.append(mp)
    obr = out_block_rows or out_rows
    return pl.pallas_call(
        body,
        grid=(SELPAD // BM_CAM,),
        in_specs=in_specs,
        out_specs=pl.BlockSpec((obr, 128), lambda g: (0 if obr == out_rows else g, 0)),
        out_shape=jax.ShapeDtypeStruct((out_rows, 128), F32),
        compiler_params=pltpu.CompilerParams(
            dimension_semantics=("arbitrary",)),
    )(*args)


def _sc_pair_sum(u, v, ia, ib):
    mesh = plsc.VectorSubcoreMesh(core_axis_name="c", subcore_axis_name="s")

    @functools.partial(
        pl.kernel,
        out_type=jax.ShapeDtypeStruct((SC_B,), F32),
        mesh=mesh,
        scratch_types=[
            pltpu.VMEM((SELPAD,), F32),
            pltpu.VMEM((SELPAD,), F32),
            pltpu.VMEM((SC_BPW,), jnp.int32),
            pltpu.VMEM((SC_BPW,), jnp.int32),
            pltpu.VMEM((SC_BPW,), F32),
        ],
        compiler_params=pltpu.CompilerParams(needs_layout_passes=False),
    )
    def k(u_hbm, v_hbm, ia_hbm, ib_hbm, o_hbm, u_v, v_v, ia_v, ib_v, o_v):
        wid = lax.axis_index("s") * 2 + lax.axis_index("c")
        base = wid * SC_BPW
        pltpu.sync_copy(u_hbm, u_v)
        pltpu.sync_copy(v_hbm, v_v)
        pltpu.sync_copy(ia_hbm.at[pl.ds(base, SC_BPW)], ia_v)
        pltpu.sync_copy(ib_hbm.at[pl.ds(base, SC_BPW)], ib_v)
        for j in range(SC_BPW // 16):
            sl = pl.ds(j * 16, 16)
            va = plsc.load_gather(u_v, [ia_v[sl]])
            vb = plsc.load_gather(v_v, [ib_v[sl]])
            o_v[sl] = va + vb
        pltpu.sync_copy(o_v, o_hbm.at[pl.ds(base, SC_BPW)])

    return k(u, v, ia, ib)


BF16 = jnp.bfloat16


def _gcn_branch(x, adj, w, b, dims):
    n, cut, bm_mm, bm2 = dims
    dinv = _dinv(adj)
    dinv_b = jnp.broadcast_to(dinv[:, None], (n, 128))
    hs = _mm(x, w, dinv_b, bm_mm, out_dtype=BF16)
    bias8 = jnp.broadcast_to(b[None, :], (8, OUT))
    full = _agg(adj, dinv_b, hs, bias8, bm2)
    return jnp.concatenate([full[:OUT], full[cut:]], axis=0)


def kernel(x_drug, x_inc, x_mrna, adj_drug, adj_inc, adj_mrna, mirna_feat,
           params, train_sample, test_sample):
    p = params
    d_map = _gcn_branch(x_drug, adj_drug, p['W_drug'], p['b_drug'],
                        GRAPH_DIMS['drug'])
    i_map = _gcn_branch(x_inc, adj_inc, p['W_inc'], p['b_inc'],
                        GRAPH_DIMS['inc'])
    m_map = _gcn_branch(x_mrna, adj_mrna, p['W_mrna'], p['b_mrna'],
                        GRAPH_DIMS['mrna'])
    maps = (d_map, i_map, m_map, mirna_feat)

    wl = p['l_c1w'][0].astype(F32)
    wg = p['g_c1w'][0].astype(F32)

    s1 = _cam_call(_p1_body, jnp.concatenate([wl, wg]), maps)[0]
    mt_l = s1[0] / CNT
    vt_l = s1[1] / CNT - mt_l * mt_l
    mt_g = s1[2] / CNT
    mt_g2 = s1[3] / CNT
    vt_g = mt_g2 - mt_g * mt_g
    a_l = p['l_g1'][0] * lax.rsqrt(vt_l + EPS)
    c_l = p['l_be1'][0] - a_l * mt_l
    a_g = p['g_g1'][0] * lax.rsqrt(vt_g + EPS)
    c_g = p['g_be1'][0] - a_g * mt_g

    scal2 = jnp.concatenate([wl, wg, jnp.stack([a_l, c_l, a_g, c_g])])
    s2 = _cam_call(_p2_body, scal2, maps)[0]
    my_l = s2[0] / CNT
    vy_l = s2[1] / CNT - my_l * my_l
    my_g = s2[2] / CNT
    vy_g = s2[3] / CNT - my_g * my_g
    w2l = p['l_c2w'][:, 0]
    w2g = p['g_c2w'][:, 0]
    sl = p['l_g2'] * w2l * lax.rsqrt(w2l * w2l * vy_l + EPS)
    sg = p['g_g2'] * w2g * lax.rsqrt(w2g * w2g * vy_g + EPS)
    q = (p['l_be2'] - sl * my_l) + (p['g_be2'] - sg * my_g)

    wc_o, t2_o, t3_o = _wc_chain(p['W1'], p['W2'], p['W3'], p['W4'])
    wc = wc_o[:, 0]
    c0 = (jnp.dot(p['b1'], t2_o[:, 0]) + jnp.dot(p['b2'], t3_o[:, 0])
          + jnp.dot(p['b3'], p['W4'][:, 0]) + p['b4'][0])
    wv = jnp.zeros((8, OUT), F32)
    wv = wv.at[0].set(wc[:OUT]).at[1].set(wc[OUT:1802])

    scal3 = jnp.concatenate([scal2, sl, sg, q])
    p3 = _cam_call(_p3_body, scal3, maps, extra_vmem=wv,
                   out_rows=SELPAD, out_block_rows=BM_CAM)
    u = p3[:, 0] + c0
    v = p3[:, 1]

    ia = jnp.concatenate([train_sample[:, 0],
                          test_sample[:, 0]]).astype(jnp.int32)
    ib = jnp.concatenate([train_sample[:, 1],
                          test_sample[:, 1]]).astype(jnp.int32)
    res = _sc_pair_sum(u, v, ia, ib)
    return (res[:16384, None], res[16384:, None])


# speedup vs baseline: 1.0138x; 1.0138x over previous
import functools

import jax
import jax.numpy as jnp
from jax import lax
from jax.experimental import pallas as pl
from jax.experimental.pallas import tpu as pltpu
from jax.experimental.pallas import tpu_sc as plsc

F32 = jnp.float32
N_NODES = 1778
OUT = 901
SELPAD = 1792
EPS = 1e-5
CNT = N_NODES * OUT

GRAPH_DIMS = {
    'drug': (2060, 1183, 416, 384),
    'inc': (2459, 1582, 496, 512),
    'mrna': (3929, 3052, 496, 512),
}
BM2 = 512
BM_CAM = 256
SC_B = 32768
SC_BPW = 1024


def _mm_body(a_ref, b_ref, d_ref, o_ref):
    a = a_ref[...].astype(BF16)
    b = b_ref[...].astype(BF16)
    acc = jnp.dot(a, b, preferred_element_type=F32)
    o_ref[...] = (acc * d_ref[:, 0:1]).astype(o_ref.dtype)


def _mm(x, w, dinv_b, bm, out_dtype=F32):
    n = x.shape[0]
    return pl.pallas_call(
        _mm_body,
        grid=(pl.cdiv(n, bm),),
        in_specs=[pl.BlockSpec((bm, n), lambda i: (i, 0)),
                  pl.BlockSpec((n, OUT), lambda i: (0, 0)),
                  pl.BlockSpec((bm, 128), lambda i: (i, 0))],
        out_specs=pl.BlockSpec((bm, OUT), lambda i: (i, 0)),
        out_shape=jax.ShapeDtypeStruct((n, OUT), out_dtype),
        compiler_params=pltpu.CompilerParams(
            dimension_semantics=("arbitrary",)),
    )(x, w, dinv_b)


def _colsum_body(a_ref, o_ref, *, nk, bk, nrows):
    k = pl.program_id(0)
    a = a_ref[...]
    rem = nrows - k * bk
    row = lax.broadcasted_iota(jnp.int32, a.shape, 0)
    a = jnp.where(row < rem, a, 0.0)
    part = jnp.broadcast_to(jnp.sum(a, axis=0, keepdims=True), o_ref.shape)

    @pl.when(k == 0)
    def _():
        o_ref[...] = part + 1.0

    @pl.when(k > 0)
    def _():
        o_ref[...] += part

    @pl.when(k == nk - 1)
    def _():
        o_ref[...] = lax.rsqrt(o_ref[...])


def _dinv(adj, bk=256):
    n = adj.shape[0]
    nk = pl.cdiv(n, bk)
    out = pl.pallas_call(
        functools.partial(_colsum_body, nk=nk, bk=bk, nrows=n),
        grid=(nk,),
        in_specs=[pl.BlockSpec((bk, n), lambda k: (k, 0))],
        out_specs=pl.BlockSpec((8, n), lambda k: (0, 0)),
        out_shape=jax.ShapeDtypeStruct((8, n), F32),
        compiler_params=pltpu.CompilerParams(
            dimension_semantics=("arbitrary",)),
    )(adj)
    return out[0]


def _agg_body(adj_ref, hs_ref, hself_ref, dc_ref, bias_ref, o_ref):
    a = adj_ref[...].astype(BF16)
    acc = lax.dot_general(
        a, hs_ref[...], (((0,), (0,)), ((), ())), preferred_element_type=F32)
    dc = dc_ref[:, 0:1]
    val = dc * (acc + hself_ref[...].astype(F32)) + bias_ref[0:1, :]
    o_ref[...] = jnp.maximum(val, 0.0).astype(BF16)


def _agg(adj, dinv_b, hs, bias8, bm2):
    n = adj.shape[0]
    return pl.pallas_call(
        _agg_body,
        grid=(pl.cdiv(n, bm2),),
        in_specs=[
            pl.BlockSpec((n, bm2), lambda i: (0, i)),
            pl.BlockSpec((n, OUT), lambda i: (0, 0)),
            pl.BlockSpec((bm2, OUT), lambda i: (i, 0)),
            pl.BlockSpec((bm2, 128), lambda i: (i, 0)),
            pl.BlockSpec((8, OUT), lambda i: (0, 0)),
        ],
        out_specs=pl.BlockSpec((bm2, OUT), lambda i: (i, 0)),
        out_shape=jax.ShapeDtypeStruct((n, OUT), BF16),
        compiler_params=pltpu.CompilerParams(
            dimension_semantics=("arbitrary",)),
    )(adj, hs, hs, dinv_b, bias8)


def _wc_body(w1_ref, w2_ref, w3_ref, w4_ref, o1_ref, o2_ref, o3_ref):
    t3 = jnp.dot(w3_ref[...], w4_ref[...], preferred_element_type=F32)
    t2 = jnp.dot(w2_ref[...], t3, preferred_element_type=F32)
    o1_ref[...] = jnp.dot(w1_ref[...], t2, preferred_element_type=F32)
    o2_ref[...] = t2
    o3_ref[...] = t3


def _wc_chain(w1, w2, w3, w4):
    full = lambda s: pl.BlockSpec(s, lambda: (0, 0))
    return pl.pallas_call(
        _wc_body,
        in_specs=[full((1802, 1024)), full((1024, 512)),
                  full((512, 64)), full((64, 1))],
        out_specs=(full((1802, 1)), full((1024, 1)), full((512, 1))),
        out_shape=(jax.ShapeDtypeStruct((1802, 1), F32),
                   jax.ShapeDtypeStruct((1024, 1), F32),
                   jax.ShapeDtypeStruct((512, 1), F32)),
    )(w1, w2, w3, w4)


def _lane_sums(vals):
    lane = lax.broadcasted_iota(jnp.int32, (8, 128), 1)
    out = jnp.zeros((8, 128), F32)
    for idx, s in enumerate(vals):
        out = out + jnp.where(lane == idx, s, 0.0)
    return out


def _chan(d_ref, i_ref, m_ref, f_ref):
    return (d_ref[...].astype(F32), i_ref[...].astype(F32),
            m_ref[...].astype(F32), f_ref[...].astype(F32))


def _p1_body(w_ref, d_ref, i_ref, m_ref, f_ref, o_ref):
    g = pl.program_id(0)
    d, i, m, f = _chan(d_ref, i_ref, m_ref, f_ref)
    tl = w_ref[0] * d + w_ref[1] * i + w_ref[2] * m + w_ref[3] * f
    tg = w_ref[4] * d + w_ref[5] * i + w_ref[6] * m + w_ref[7] * f
    row = g * BM_CAM + lax.broadcasted_iota(jnp.int32, tl.shape, 0)
    ok = row < N_NODES
    tl = jnp.where(ok, tl, 0.0)
    tg = jnp.where(ok, tg, 0.0)
    part = _lane_sums([jnp.sum(tl), jnp.sum(tl * tl),
                       jnp.sum(tg), jnp.sum(tg * tg)])

    @pl.when(g == 0)
    def _():
        o_ref[...] = part

    @pl.when(g > 0)
    def _():
        o_ref[...] += part


def _p2_body(s_ref, d_ref, i_ref, m_ref, f_ref, o_ref):
    g = pl.program_id(0)
    d, i, m, f = _chan(d_ref, i_ref, m_ref, f_ref)
    tl = s_ref[0] * d + s_ref[1] * i + s_ref[2] * m + s_ref[3] * f
    tg = s_ref[4] * d + s_ref[5] * i + s_ref[6] * m + s_ref[7] * f
    yl = jnp.maximum(s_ref[8] * tl + s_ref[9], 0.0)
    yg = jnp.maximum(s_ref[10] * tg + s_ref[11], 0.0)
    row = g * BM_CAM + lax.broadcasted_iota(jnp.int32, yl.shape, 0)
    ok = row < N_NODES
    yl = jnp.where(ok, yl, 0.0)
    yg = jnp.where(ok, yg, 0.0)
    part = _lane_sums([jnp.sum(yl), jnp.sum(yl * yl),
                       jnp.sum(yg), jnp.sum(yg * yg)])

    @pl.when(g == 0)
    def _():
        o_ref[...] = part

    @pl.when(g > 0)
    def _():
        o_ref[...] += part


def _p3_body(s_ref, wv_ref, d_ref, i_ref, m_ref, f_ref, o_ref):
    chans = _chan(d_ref, i_ref, m_ref, f_ref)
    d, i, m, f = chans
    tl = s_ref[0] * d + s_ref[1] * i + s_ref[2] * m + s_ref[3] * f
    tg = s_ref[4] * d + s_ref[5] * i + s_ref[6] * m + s_ref[7] * f
    yl = jnp.maximum(s_ref[8] * tl + s_ref[9], 0.0)
    yg = jnp.maximum(s_ref[10] * tg + s_ref[11], 0.0)
    sx = jnp.zeros_like(tl)
    for c in range(4):
        att = jax.nn.sigmoid(s_ref[12 + c] * yl + s_ref[16 + c] * yg
                             + s_ref[20 + c])
        sx = sx + chans[c] * att
    sx = sx * 0.25
    u = jnp.sum(sx * wv_ref[0:1, :], axis=1)
    v = jnp.sum(sx * wv_ref[1:2, :], axis=1)
    lane = lax.broadcasted_iota(jnp.int32, o_ref.shape, 1)
    o_ref[...] = (jnp.where(lane == 0, u[:, None], 0.0)
                  + jnp.where(lane == 1, v[:, None], 0.0))


def _cam_call(body, scal, maps, extra_vmem=None, out_rows=8,
              out_block_rows=None):
    n_scal = scal.shape[0]
    in_specs = [pl.BlockSpec(memory_space=pltpu.SMEM)]
    args = [scal]
    if extra_vmem is not None:
        in_specs.append(pl.BlockSpec((8, OUT), lambda g: (0, 0)))
        args.append(extra_vmem)
    for mp in maps:
        in_specs.append(pl.BlockSpec((BM_CAM, OUT), lambda g: (g, 0)))
        args.append(mp)
    obr = out_block_rows or out_rows
    return pl.pallas_call(
        body,
        grid=(SELPAD // BM_CAM,),
        in_specs=in_specs,
        out_specs=pl.BlockSpec((obr, 128), lambda g: (0 if obr == out_rows else g, 0)),
        out_shape=jax.ShapeDtypeStruct((out_rows, 128), F32),
        compiler_params=pltpu.CompilerParams(
            dimension_semantics=("arbitrary",)),
    )(*args)


def _sc_pair_sum(u, v, ia, ib):
    mesh = plsc.VectorSubcoreMesh(core_axis_name="c", subcore_axis_name="s")

    @functools.partial(
        pl.kernel,
        out_type=jax.ShapeDtypeStruct((SC_B,), F32),
        mesh=mesh,
        scratch_types=[
            pltpu.VMEM((SELPAD,), F32),
            pltpu.VMEM((SELPAD,), F32),
            pltpu.VMEM((SC_BPW,), jnp.int32),
            pltpu.VMEM((SC_BPW,), jnp.int32),
            pltpu.VMEM((SC_BPW,), F32),
        ],
        compiler_params=pltpu.CompilerParams(needs_layout_passes=False),
    )
    def k(u_hbm, v_hbm, ia_hbm, ib_hbm, o_hbm, u_v, v_v, ia_v, ib_v, o_v):
        wid = lax.axis_index("s") * 2 + lax.axis_index("c")
        base = wid * SC_BPW
        pltpu.sync_copy(u_hbm, u_v)
        pltpu.sync_copy(v_hbm, v_v)
        pltpu.sync_copy(ia_hbm.at[pl.ds(base, SC_BPW)], ia_v)
        pltpu.sync_copy(ib_hbm.at[pl.ds(base, SC_BPW)], ib_v)
        for j in range(SC_BPW // 16):
            sl = pl.ds(j * 16, 16)
            va = plsc.load_gather(u_v, [ia_v[sl]])
            vb = plsc.load_gather(v_v, [ib_v[sl]])
            o_v[sl] = va + vb
        pltpu.sync_copy(o_v, o_hbm.at[pl.ds(base, SC_BPW)])

    return k(u, v, ia, ib)


BF16 = jnp.bfloat16


def _gcn_branch(x, adj, w, b, dims):
    n, cut, bm_mm, bm2 = dims
    dinv = _dinv(adj)
    dinv_b = jnp.broadcast_to(dinv[:, None], (n, 128))
    hs = _mm(x, w, dinv_b, bm_mm, out_dtype=BF16)
    bias8 = jnp.broadcast_to(b[None, :], (8, OUT))
    full = _agg(adj, dinv_b, hs, bias8, bm2)
    return jnp.concatenate([full[:OUT], full[cut:]], axis=0)


def kernel(x_drug, x_inc, x_mrna, adj_drug, adj_inc, adj_mrna, mirna_feat,
           params, train_sample, test_sample):
    p = params
    d_map = _gcn_branch(x_drug, adj_drug, p['W_drug'], p['b_drug'],
                        GRAPH_DIMS['drug'])
    i_map = _gcn_branch(x_inc, adj_inc, p['W_inc'], p['b_inc'],
                        GRAPH_DIMS['inc'])
    m_map = _gcn_branch(x_mrna, adj_mrna, p['W_mrna'], p['b_mrna'],
                        GRAPH_DIMS['mrna'])
    maps = (d_map, i_map, m_map, mirna_feat)

    wl = p['l_c1w'][0].astype(F32)
    wg = p['g_c1w'][0].astype(F32)

    s1 = _cam_call(_p1_body, jnp.concatenate([wl, wg]), maps)[0]
    mt_l = s1[0] / CNT
    vt_l = s1[1] / CNT - mt_l * mt_l
    mt_g = s1[2] / CNT
    mt_g2 = s1[3] / CNT
    vt_g = mt_g2 - mt_g * mt_g
    a_l = p['l_g1'][0] * lax.rsqrt(vt_l + EPS)
    c_l = p['l_be1'][0] - a_l * mt_l
    a_g = p['g_g1'][0] * lax.rsqrt(vt_g + EPS)
    c_g = p['g_be1'][0] - a_g * mt_g

    scal2 = jnp.concatenate([wl, wg, jnp.stack([a_l, c_l, a_g, c_g])])
    s2 = _cam_call(_p2_body, scal2, maps)[0]
    my_l = s2[0] / CNT
    vy_l = s2[1] / CNT - my_l * my_l
    my_g = s2[2] / CNT
    vy_g = s2[3] / CNT - my_g * my_g
    w2l = p['l_c2w'][:, 0]
    w2g = p['g_c2w'][:, 0]
    sl = p['l_g2'] * w2l * lax.rsqrt(w2l * w2l * vy_l + EPS)
    sg = p['g_g2'] * w2g * lax.rsqrt(w2g * w2g * vy_g + EPS)
    q = (p['l_be2'] - sl * my_l) + (p['g_be2'] - sg * my_g)

    wc_o, t2_o, t3_o = _wc_chain(p['W1'], p['W2'], p['W3'], p['W4'])
    wc = wc_o[:, 0]
    c0 = (jnp.dot(p['b1'], t2_o[:, 0]) + jnp.dot(p['b2'], t3_o[:, 0])
          + jnp.dot(p['b3'], p['W4'][:, 0]) + p['b4'][0])
    wv = jnp.zeros((8, OUT), F32)
    wv = wv.at[0].set(wc[:OUT]).at[1].set(wc[OUT:1802])

    scal3 = jnp.concatenate([scal2, sl, sg, q])
    p3 = _cam_call(_p3_body, scal3, maps, extra_vmem=wv,
                   out_rows=SELPAD, out_block_rows=BM_CAM)
    u = p3[:, 0] + c0
    v = p3[:, 1]

    ia = jnp.concatenate([train_sample[:, 0],
                          test_sample[:, 0]]).astype(jnp.int32)
    ib = jnp.concatenate([train_sample[:, 1],
                          test_sample[:, 1]]).astype(jnp.int32)
    res = _sc_pair_sum(u, v, ia, ib)
    return (res[:16384, None], res[16384:, None])
